# Initial kernel scaffold; baseline (speedup 1.0000x reference)
import jax
import jax.numpy as jnp
from jax.experimental import pallas as pl


def kernel(q_title_tensors, q_body_tensors, candidate_title_tensors, candidate_body_tensors, embeddings):
    raise NotImplementedError("write your pallas kernel here")


# speedup vs baseline: 7.4253x; 7.4253x over previous
import jax
import jax.numpy as jnp
from jax import lax
from jax.experimental import pallas as pl
from jax.experimental.pallas import tpu as pltpu
from jax.experimental.pallas import tpu_sc as plsc

VOCAB = 100000
EMBED = 128
B = 1024
L = 50
NC = 20
EPS = 1e-8

NUM_CORES = 2
NUM_SUBCORES = 16
NW = NUM_CORES * NUM_SUBCORES
BPW = B // NW
LANES = 16
NCH = EMBED // LANES
SEG = 2 * L


def _pool_body(qt_hbm, qb_hbm, ct_hbm, cb_hbm, table_hbm,
               dot_hbm, qq_hbm, cc_hbm,
               qt_v, qb_v, ctb_v, cbb_v, rows_v, dot_v, qq_v, sem, sem2):
    wid = lax.axis_index("s") * NUM_CORES + lax.axis_index("c")
    base = wid * BPW

    pltpu.sync_copy(qt_hbm.at[pl.ds(base, BPW)], qt_v)
    pltpu.sync_copy(qb_hbm.at[pl.ds(base, BPW)], qb_v)

    def gather_segment(idx_title, idx_body):
        cp1 = pltpu.make_async_copy(table_hbm.at[idx_title],
                                    rows_v.at[pl.ds(0, L)], sem)
        cp2 = pltpu.make_async_copy(table_hbm.at[idx_body],
                                    rows_v.at[pl.ds(L, L)], sem)
        cp1.start()
        cp2.start()
        cp1.wait()
        cp2.wait()

    def reduce_rows():
        def body(r, accs):
            return tuple(accs[c] + rows_v[r, pl.ds(c * LANES, LANES)]
                         for c in range(NCH))
        zeros = tuple(jnp.zeros((LANES,), jnp.float32) for _ in range(NCH))
        return lax.fori_loop(0, SEG, body, zeros)

    def lane_sum(v):
        dnums = lax.GatherDimensionNumbers(
            offset_dims=(), collapsed_slice_dims=(0,), start_index_map=(0,))
        for k in (8, 4, 2, 1):
            perm = lax.iota(jnp.int32, LANES) ^ k
            v = v + lax.gather(v, perm[:, None], dnums, slice_sizes=(1,),
                               mode=lax.GatherScatterMode.PROMISE_IN_BOUNDS)
        return v

    def per_b(b, _):
        st1 = pltpu.make_async_copy(ct_hbm.at[:, base + b, :], ctb_v, sem2)
        st2 = pltpu.make_async_copy(cb_hbm.at[:, base + b, :], cbb_v, sem2)
        st1.start()
        st2.start()

        gather_segment(qt_v.at[b], qb_v.at[b])
        q = reduce_rows()
        qq = jnp.zeros((LANES,), jnp.float32)
        for c in range(NCH):
            qq = qq + q[c] * q[c]
        qq_v[pl.ds(b * LANES, LANES)] = lane_sum(qq)

        st1.wait()
        st2.wait()

        def per_nc(nc, _):
            gather_segment(ctb_v.at[nc], cbb_v.at[nc])
            cvec = reduce_rows()
            cc = jnp.zeros((LANES,), jnp.float32)
            dt = jnp.zeros((LANES,), jnp.float32)
            for c in range(NCH):
                cc = cc + cvec[c] * cvec[c]
                dt = dt + q[c] * cvec[c]
            off = (b * NC + nc) * LANES
            dot_v[pl.ds(off, LANES)] = lane_sum(dt)
            dot_v[pl.ds(BPW * NC * LANES + off, LANES)] = lane_sum(cc)
            return 0

        lax.fori_loop(0, NC, per_nc, 0)
        return 0

    lax.fori_loop(0, BPW, per_b, 0)

    n = BPW * NC * LANES
    pltpu.sync_copy(dot_v.at[pl.ds(0, n)], dot_hbm.at[pl.ds(base * NC * LANES, n)])
    pltpu.sync_copy(dot_v.at[pl.ds(n, n)], cc_hbm.at[pl.ds(base * NC * LANES, n)])
    pltpu.sync_copy(qq_v, qq_hbm.at[pl.ds(base * LANES, BPW * LANES)])


_pool = pl.kernel(
    _pool_body,
    out_type=(
        jax.ShapeDtypeStruct((B * NC * LANES,), jnp.float32),
        jax.ShapeDtypeStruct((B * LANES,), jnp.float32),
        jax.ShapeDtypeStruct((B * NC * LANES,), jnp.float32),
    ),
    mesh=plsc.VectorSubcoreMesh(core_axis_name="c", subcore_axis_name="s",
                                num_cores=NUM_CORES,
                                num_subcores=NUM_SUBCORES),
    scratch_types=[
        pltpu.VMEM((BPW, L), jnp.int32),
        pltpu.VMEM((BPW, L), jnp.int32),
        pltpu.VMEM((NC, L), jnp.int32),
        pltpu.VMEM((NC, L), jnp.int32),
        pltpu.VMEM((SEG, EMBED), jnp.float32),
        pltpu.VMEM((2 * BPW * NC * LANES,), jnp.float32),
        pltpu.VMEM((BPW * LANES,), jnp.float32),
        pltpu.SemaphoreType.DMA,
        pltpu.SemaphoreType.DMA,
    ],
)


def _cos_body(dot_ref, qq_ref, cc_ref, o_ref):
    inv = jnp.float32(1.0 / SEG)
    qn = jnp.maximum(jnp.sqrt(qq_ref[...]) * inv, EPS)
    cn = jnp.maximum(jnp.sqrt(cc_ref[...]) * inv, EPS)
    o_ref[...] = (dot_ref[...] * (inv * inv)) / (qn * cn)


_cos = pl.pallas_call(
    _cos_body,
    out_shape=jax.ShapeDtypeStruct((B, NC), jnp.float32),
)


def kernel(q_title_tensors, q_body_tensors, candidate_title_tensors,
           candidate_body_tensors, embeddings):
    qt = q_title_tensors.astype(jnp.int32)
    qb = q_body_tensors.astype(jnp.int32)
    ct = candidate_title_tensors.astype(jnp.int32)
    cb = candidate_body_tensors.astype(jnp.int32)
    dots, qq, cc = _pool(qt, qb, ct, cb, embeddings)
    dots = dots.reshape(B, NC, LANES)[:, :, 0]
    cc = cc.reshape(B, NC, LANES)[:, :, 0]
    qq = qq.reshape(B, LANES)[:, :1]
    return _cos(dots, qq, cc)


# speedup vs baseline: 19.4951x; 2.6255x over previous
import jax
import jax.numpy as jnp
from jax import lax
from jax.experimental import pallas as pl
from jax.experimental.pallas import tpu as pltpu
from jax.experimental.pallas import tpu_sc as plsc

VOCAB = 100000
EMBED = 128
B = 1024
L = 50
NC = 20
EPS = 1e-8

NUM_CORES = 2
NUM_SUBCORES = 16
NW = NUM_CORES * NUM_SUBCORES
BPW = B // NW
LANES = 16
NCH = EMBED // LANES
SEG = 2 * L
NSEG = NC + 1
TOT = BPW * NSEG
DEPTH = 4

DOT0 = 0
CC0 = BPW * NC * LANES
QQ0 = 2 * BPW * NC * LANES
TRASH = QQ0 + BPW * LANES
RES_N = TRASH + LANES


def _pool_body(qt_hbm, qb_hbm, ct_hbm, cb_hbm, table_hbm,
               dot_hbm, qq_hbm, cc_hbm,
               qt_v, qb_v, ctb_v, cbb_v, rows0, rows1, rows2, rows3, res_v,
               sg0, sg1, sg2, sg3, sem_s):
    wid = lax.axis_index("s") * NUM_CORES + lax.axis_index("c")
    base = wid * BPW
    rows = (rows0, rows1, rows2, rows3)
    sems = (sg0, sg1, sg2, sg3)

    def stage_copies(bq):
        p = lax.rem(bq, 3)
        return (
            pltpu.make_async_copy(ct_hbm.at[:, base + bq, :],
                                  ctb_v.at[p], sem_s),
            pltpu.make_async_copy(cb_hbm.at[:, base + bq, :],
                                  cbb_v.at[p], sem_s),
        )

    def start_gather(idx_title, idx_body, h):
        pltpu.make_async_copy(table_hbm.at[idx_title],
                              rows[h].at[pl.ds(0, L)], sems[h]).start()
        pltpu.make_async_copy(table_hbm.at[idx_body],
                              rows[h].at[pl.ds(L, L)], sems[h]).start()

    def start_gather_seg(s, h):
        b = lax.div(s, NSEG)
        j = s - b * NSEG
        p = lax.rem(b, 3)

        @pl.when(j == 0)
        def _():
            start_gather(qt_v.at[b], qb_v.at[b], h)

        @pl.when(j != 0)
        def _():
            start_gather(ctb_v.at[p, j - 1], cbb_v.at[p, j - 1], h)

    def drain_gather(h):
        pltpu.make_async_copy(table_hbm.at[qt_v.at[0]],
                              rows[h].at[pl.ds(0, L)], sems[h]).wait()
        pltpu.make_async_copy(table_hbm.at[qt_v.at[0]],
                              rows[h].at[pl.ds(L, L)], sems[h]).wait()

    def reduce_rows(h):
        rref = rows[h]

        def body(r, accs):
            return tuple(accs[c] + rref[r, pl.ds(c * LANES, LANES)]
                         for c in range(NCH))
        zeros = tuple(jnp.zeros((LANES,), jnp.float32) for _ in range(NCH))
        return lax.fori_loop(0, SEG, body, zeros, unroll=4)

    def lane_sum(v):
        dnums = lax.GatherDimensionNumbers(
            offset_dims=(), collapsed_slice_dims=(0,), start_index_map=(0,))
        for k in (8, 4, 2, 1):
            perm = lax.iota(jnp.int32, LANES) ^ k
            v = v + lax.gather(v, perm[:, None], dnums, slice_sizes=(1,),
                               mode=lax.GatherScatterMode.PROMISE_IN_BOUNDS)
        return v

    pltpu.sync_copy(qt_hbm.at[pl.ds(base, BPW)], qt_v)
    pltpu.sync_copy(qb_hbm.at[pl.ds(base, BPW)], qb_v)
    for cp in stage_copies(jnp.int32(0)):
        cp.start()
        cp.wait()
    for h in range(DEPTH):
        start_gather_seg(jnp.int32(h), h)

    def iteration(i, q):
        s0 = i * DEPTH
        for h in range(DEPTH):
            s = s0 + h
            b = lax.div(s, NSEG)
            j = s - b * NSEG
            drain_gather(h)

            @pl.when(jnp.logical_and(j == 0, b + 1 < BPW))
            def _():
                for cp in stage_copies(b + 1):
                    cp.start()

            rr = reduce_rows(h)

            s4 = jnp.minimum(s + DEPTH, TOT - 1)
            b4 = lax.div(s4, NSEG)
            j4 = s4 - b4 * NSEG

            @pl.when(j4 == 0)
            def _():
                for cp in stage_copies(b4):
                    cp.wait()

            start_gather_seg(s4, h)

            q = tuple(jnp.where(j == 0, rr[c], q[c]) for c in range(NCH))
            ss = jnp.zeros((LANES,), jnp.float32)
            dt = jnp.zeros((LANES,), jnp.float32)
            for c in range(NCH):
                ss = ss + rr[c] * rr[c]
                dt = dt + q[c] * rr[c]
            pair = (b * NC + (j - 1)) * LANES
            ss_off = jnp.where(j == 0, QQ0 + b * LANES, CC0 + pair)
            dt_off = jnp.where(j == 0, TRASH, DOT0 + pair)
            res_v[pl.ds(ss_off, LANES)] = lane_sum(ss)
            res_v[pl.ds(dt_off, LANES)] = lane_sum(dt)
        return q

    zeros = tuple(jnp.zeros((LANES,), jnp.float32) for _ in range(NCH))
    lax.fori_loop(0, TOT // DEPTH, iteration, zeros)

    for h in range(DEPTH):
        drain_gather(h)

    n = BPW * NC * LANES
    pltpu.sync_copy(res_v.at[pl.ds(DOT0, n)],
                    dot_hbm.at[pl.ds(base * NC * LANES, n)])
    pltpu.sync_copy(res_v.at[pl.ds(CC0, n)],
                    cc_hbm.at[pl.ds(base * NC * LANES, n)])
    pltpu.sync_copy(res_v.at[pl.ds(QQ0, BPW * LANES)],
                    qq_hbm.at[pl.ds(base * LANES, BPW * LANES)])


_pool = pl.kernel(
    _pool_body,
    out_type=(
        jax.ShapeDtypeStruct((B * NC * LANES,), jnp.float32),
        jax.ShapeDtypeStruct((B * LANES,), jnp.float32),
        jax.ShapeDtypeStruct((B * NC * LANES,), jnp.float32),
    ),
    mesh=plsc.VectorSubcoreMesh(core_axis_name="c", subcore_axis_name="s",
                                num_cores=NUM_CORES,
                                num_subcores=NUM_SUBCORES),
    scratch_types=(
        [pltpu.VMEM((BPW, L), jnp.int32)] * 2 +
        [pltpu.VMEM((3, NC, L), jnp.int32)] * 2 +
        [pltpu.VMEM((SEG, EMBED), jnp.float32)] * DEPTH +
        [pltpu.VMEM((RES_N,), jnp.float32)] +
        [pltpu.SemaphoreType.DMA] * (DEPTH + 1)
    ),
)


def _cos_body(dot_ref, qq_ref, cc_ref, o_ref):
    inv = jnp.float32(1.0 / SEG)
    qn = jnp.maximum(jnp.sqrt(qq_ref[...]) * inv, EPS)
    cn = jnp.maximum(jnp.sqrt(cc_ref[...]) * inv, EPS)
    o_ref[...] = (dot_ref[...] * (inv * inv)) / (qn * cn)


_cos = pl.pallas_call(
    _cos_body,
    out_shape=jax.ShapeDtypeStruct((B, NC), jnp.float32),
)


def kernel(q_title_tensors, q_body_tensors, candidate_title_tensors,
           candidate_body_tensors, embeddings):
    qt = q_title_tensors.astype(jnp.int32)
    qb = q_body_tensors.astype(jnp.int32)
    ct = candidate_title_tensors.astype(jnp.int32)
    cb = candidate_body_tensors.astype(jnp.int32)
    dots, qq, cc = _pool(qt, qb, ct, cb, embeddings)
    dots = dots.reshape(B, NC, LANES)[:, :, 0]
    cc = cc.reshape(B, NC, LANES)[:, :, 0]
    qq = qq.reshape(B, LANES)[:, :1]
    return _cos(dots, qq, cc)


# speedup vs baseline: 21.2806x; 1.0916x over previous
import jax
import jax.numpy as jnp
from jax import lax
from jax.experimental import pallas as pl
from jax.experimental.pallas import tpu as pltpu
from jax.experimental.pallas import tpu_sc as plsc

VOCAB = 100000
EMBED = 128
B = 1024
L = 50
NC = 20
EPS = 1e-8

NUM_CORES = 2
NUM_SUBCORES = 16
NW = NUM_CORES * NUM_SUBCORES
BPW = B // NW
LANES = 16
NCH = EMBED // LANES
SEG = 2 * L
NSEG = NC + 1
TOT = BPW * NSEG
DEPTH = 6

DOT0 = 0
CC0 = BPW * NC
QQ0 = 2 * BPW * NC
RES_N = QQ0 + BPW


def _pool_body(qt_hbm, qb_hbm, ct_hbm, cb_hbm, table_hbm,
               dot_hbm, qq_hbm, cc_hbm,
               qt_v, qb_v, ctb_v, cbb_v,
               rows0, rows1, rows2, rows3, rows4, rows5, res_v,
               sg0, sg1, sg2, sg3, sg4, sg5, sem_s):
    wid = lax.axis_index("s") * NUM_CORES + lax.axis_index("c")
    base = wid * BPW
    rows = (rows0, rows1, rows2, rows3, rows4, rows5)
    sems = (sg0, sg1, sg2, sg3, sg4, sg5)

    def stage_copies(bq):
        p = lax.rem(bq, 3)
        return (
            pltpu.make_async_copy(ct_hbm.at[:, base + bq, :],
                                  ctb_v.at[p], sem_s),
            pltpu.make_async_copy(cb_hbm.at[:, base + bq, :],
                                  cbb_v.at[p], sem_s),
        )

    def start_gather(idx_title, idx_body, h):
        pltpu.make_async_copy(table_hbm.at[idx_title],
                              rows[h].at[pl.ds(0, L)], sems[h]).start()
        pltpu.make_async_copy(table_hbm.at[idx_body],
                              rows[h].at[pl.ds(L, L)], sems[h]).start()

    def start_gather_seg(s, h):
        b = lax.div(s, NSEG)
        j = s - b * NSEG
        p = lax.rem(b, 3)

        @pl.when(j == 0)
        def _():
            start_gather(qt_v.at[b], qb_v.at[b], h)

        @pl.when(j != 0)
        def _():
            start_gather(ctb_v.at[p, j - 1], cbb_v.at[p, j - 1], h)

    def drain_gather(h):
        pltpu.make_async_copy(table_hbm.at[qt_v.at[0]],
                              rows[h].at[pl.ds(0, L)], sems[h]).wait()
        pltpu.make_async_copy(table_hbm.at[qt_v.at[0]],
                              rows[h].at[pl.ds(L, L)], sems[h]).wait()

    def reduce_rows(h):
        rref = rows[h]

        def body(r, accs):
            return tuple(accs[c] + rref[r, pl.ds(c * LANES, LANES)]
                         for c in range(NCH))
        zeros = tuple(jnp.zeros((LANES,), jnp.float32) for _ in range(NCH))
        return lax.fori_loop(0, SEG, body, zeros, unroll=4)

    lanes_iota = lax.iota(jnp.int32, LANES)

    def lane_sum(v):
        dnums = lax.GatherDimensionNumbers(
            offset_dims=(), collapsed_slice_dims=(0,), start_index_map=(0,))
        for k in (8, 4, 2, 1):
            perm = lanes_iota ^ k
            v = v + lax.gather(v, perm[:, None], dnums, slice_sizes=(1,),
                               mode=lax.GatherScatterMode.PROMISE_IN_BOUNDS)
        return v

    pltpu.sync_copy(qt_hbm.at[pl.ds(base, BPW)], qt_v)
    pltpu.sync_copy(qb_hbm.at[pl.ds(base, BPW)], qb_v)
    for cp in stage_copies(jnp.int32(0)):
        cp.start()
        cp.wait()
    for h in range(DEPTH):
        start_gather_seg(jnp.int32(h), h)

    def iteration(i, carry):
        q, pdot, pcc, pqq = carry
        s0 = i * DEPTH
        for h in range(DEPTH):
            s = s0 + h
            b = lax.div(s, NSEG)
            j = s - b * NSEG
            drain_gather(h)

            @pl.when(jnp.logical_and(j == 0, b + 1 < BPW))
            def _():
                for cp in stage_copies(b + 1):
                    cp.start()

            rr = reduce_rows(h)

            s4 = jnp.minimum(s + DEPTH, TOT - 1)
            b4 = lax.div(s4, NSEG)
            j4 = s4 - b4 * NSEG

            @pl.when(j4 == 0)
            def _():
                for cp in stage_copies(b4):
                    cp.wait()

            start_gather_seg(s4, h)

            q = tuple(jnp.where(j == 0, rr[c], q[c]) for c in range(NCH))
            ss = jnp.zeros((LANES,), jnp.float32)
            dt = jnp.zeros((LANES,), jnp.float32)
            for c in range(NCH):
                ss = ss + rr[c] * rr[c]
                dt = dt + q[c] * rr[c]
            ss = lane_sum(ss)
            dt = lane_sum(dt)

            pair = b * NC + (j - 1)
            slot_c = jnp.where(j != 0, pair & (LANES - 1), -1)
            mpair = lanes_iota == slot_c
            pdot = jnp.where(mpair, dt, pdot)
            pcc = jnp.where(mpair, ss, pcc)
            slot_q = jnp.where(j == 0, b & (LANES - 1), -1)
            pqq = jnp.where(lanes_iota == slot_q, ss, pqq)
            goff = jnp.maximum(pair, 0) & ~(LANES - 1)
            res_v[pl.ds(DOT0 + goff, LANES)] = pdot
            res_v[pl.ds(CC0 + goff, LANES)] = pcc
            res_v[pl.ds(QQ0 + (b & ~(LANES - 1)), LANES)] = pqq
        return q, pdot, pcc, pqq

    zeros = tuple(jnp.zeros((LANES,), jnp.float32) for _ in range(NCH))
    zv = jnp.zeros((LANES,), jnp.float32)
    lax.fori_loop(0, TOT // DEPTH, iteration, (zeros, zv, zv, zv))

    for h in range(DEPTH):
        drain_gather(h)

    n = BPW * NC
    pltpu.sync_copy(res_v.at[pl.ds(DOT0, n)], dot_hbm.at[pl.ds(base * NC, n)])
    pltpu.sync_copy(res_v.at[pl.ds(CC0, n)], cc_hbm.at[pl.ds(base * NC, n)])
    pltpu.sync_copy(res_v.at[pl.ds(QQ0, BPW)], qq_hbm.at[pl.ds(base, BPW)])


_pool = pl.kernel(
    _pool_body,
    out_type=(
        jax.ShapeDtypeStruct((B * NC,), jnp.float32),
        jax.ShapeDtypeStruct((B,), jnp.float32),
        jax.ShapeDtypeStruct((B * NC,), jnp.float32),
    ),
    mesh=plsc.VectorSubcoreMesh(core_axis_name="c", subcore_axis_name="s",
                                num_cores=NUM_CORES,
                                num_subcores=NUM_SUBCORES),
    scratch_types=(
        [pltpu.VMEM((BPW, L), jnp.int32)] * 2 +
        [pltpu.VMEM((3, NC, L), jnp.int32)] * 2 +
        [pltpu.VMEM((SEG, EMBED), jnp.float32)] * DEPTH +
        [pltpu.VMEM((RES_N,), jnp.float32)] +
        [pltpu.SemaphoreType.DMA] * (DEPTH + 1)
    ),
)


def _cos_body(dot_ref, qq_ref, cc_ref, o_ref):
    inv = jnp.float32(1.0 / SEG)
    qn = jnp.maximum(jnp.sqrt(qq_ref[...]) * inv, EPS)
    cn = jnp.maximum(jnp.sqrt(cc_ref[...]) * inv, EPS)
    o_ref[...] = (dot_ref[...] * (inv * inv)) / (qn * cn)


_cos = pl.pallas_call(
    _cos_body,
    out_shape=jax.ShapeDtypeStruct((B, NC), jnp.float32),
)


def kernel(q_title_tensors, q_body_tensors, candidate_title_tensors,
           candidate_body_tensors, embeddings):
    qt = q_title_tensors.astype(jnp.int32)
    qb = q_body_tensors.astype(jnp.int32)
    ct = candidate_title_tensors.astype(jnp.int32)
    cb = candidate_body_tensors.astype(jnp.int32)
    dots, qq, cc = _pool(qt, qb, ct, cb, embeddings)
    return _cos(dots.reshape(B, NC), qq.reshape(B, 1), cc.reshape(B, NC))
